# Initial kernel scaffold; baseline (speedup 1.0000x reference)
import jax
import jax.numpy as jnp
from jax.experimental import pallas as pl


def kernel(f_BNC, base, proj_w, proj_b):
    raise NotImplementedError("write your pallas kernel here")


# speedup vs baseline: 1.0039x; 1.0039x over previous
import jax
import jax.numpy as jnp
from jax.experimental import pallas as pl

B, N, C, K = 8, 1024, 256, 8192
SCALES = [2 ** i for i in range(11)]


def _dist_argmin_body(rest_ref, rsq_ref, embT_ref, esq_ref, idx_ref, *, n_chunks):
    rest = rest_ref[...]
    rsq = rsq_ref[...]
    W = K // n_chunks
    acc_val = None
    acc_idx = None
    for c in range(n_chunks):
        m = jnp.dot(rest, embT_ref[:, c * W:(c + 1) * W],
                    preferred_element_type=jnp.float32)
        d = (rsq + esq_ref[:, c * W:(c + 1) * W]) - 2.0 * m
        cmin = jnp.min(d, axis=1, keepdims=True)
        iota = jax.lax.broadcasted_iota(jnp.int32, d.shape, 1)
        cidx = jnp.min(jnp.where(d == cmin, iota, K), axis=1, keepdims=True) + c * W
        if acc_val is None:
            acc_val, acc_idx = cmin, cidx
        else:
            take = (cmin < acc_val) | ((cmin == acc_val) & (cidx < acc_idx))
            acc_idx = jnp.where(take, cidx, acc_idx)
            acc_val = jnp.where(take, cmin, acc_val)
        if n_chunks > 1:
            acc_val = acc_val.astype(jnp.bfloat16).astype(jnp.float32)
    idx_ref[...] = acc_idx


def _dist_argmin(rest, rsq, embT, esq, n_chunks=1):
    import functools
    R = rest.shape[0]
    Rb = min(R, 256)
    grid = (R // Rb,)
    return pl.pallas_call(
        functools.partial(_dist_argmin_body, n_chunks=n_chunks),
        grid=grid,
        in_specs=[
            pl.BlockSpec((Rb, C), lambda r: (r, 0)),
            pl.BlockSpec((Rb, 1), lambda r: (r, 0)),
            pl.BlockSpec((C, K), lambda r: (0, 0)),
            pl.BlockSpec((1, K), lambda r: (0, 0)),
        ],
        out_specs=pl.BlockSpec((Rb, 1), lambda r: (r, 0)),
        out_shape=jax.ShapeDtypeStruct((R, 1), jnp.int32),
    )(rest, rsq, embT, esq)


def _area_down_(x, pn):
    b, c, n = x.shape
    return x.reshape(b, c, pn, n // pn).mean(axis=-1)


def _linear_up_(x, out_size):
    n = x.shape[-1]
    if n == out_size:
        return x
    scale = n / out_size
    coords = (jnp.arange(out_size, dtype=jnp.float32) + 0.5) * scale - 0.5
    coords = jnp.clip(coords, 0.0, float(n - 1))
    lo = jnp.floor(coords).astype(jnp.int32)
    hi = jnp.minimum(lo + 1, n - 1)
    w = (coords - lo.astype(jnp.float32))[None, None, :]
    return jnp.take(x, lo, axis=-1) * (1.0 - w) + jnp.take(x, hi, axis=-1) * w


def kernel(f_BNC, base, proj_w, proj_b):
    embedding = base @ proj_w.T + proj_b
    emb_ng = embedding
    embT = emb_ng.T
    esq = jnp.sum(emb_ng ** 2, axis=1).reshape(1, K)
    f_BCN = jnp.transpose(f_BNC, (0, 2, 1))
    f_ng = f_BCN
    f_rest = f_ng
    f_hat = jnp.zeros_like(f_ng)
    SN = len(SCALES)
    commit = jnp.float32(0.0)
    qlat = jnp.float32(0.0)
    for pn in SCALES:
        rest_NC = jnp.transpose(_area_down_(f_rest, pn), (0, 2, 1)).reshape(-1, C)
        rsq = jnp.sum(rest_NC ** 2, axis=1, keepdims=True)
        idx = _dist_argmin(rest_NC, rsq, embT, esq,
                           n_chunks=4 if pn == N else 1)[:, 0]
        h_NC = jnp.take(embedding, idx, axis=0)
        h_BnC = h_NC.reshape(B, pn, C)
        h_BCn = _linear_up_(jnp.transpose(h_BnC, (0, 2, 1)), N)
        f_hat = f_hat + h_BCn
        f_rest = f_rest - h_BCn
        commit = commit + 0.25 * jnp.mean((f_hat - f_BCN) ** 2)
        qlat = qlat + jnp.mean((f_hat - f_ng) ** 2)
    commit = commit / SN
    qlat = qlat / SN
    f_hat_out = f_hat - f_ng + f_BCN
    return (jnp.transpose(f_hat_out, (0, 2, 1)), commit, qlat, 0.0)
